# baseline (speedup 1.0000x reference)
import jax
import jax.numpy as jnp
from jax import lax
from jax.experimental import pallas as pl
from jax.experimental.pallas import tpu as pltpu
from jax.experimental.pallas import tpu_sc as plsc

_B, _S, _H = 4, 2048, 4096
_NC, _NS = 2, 16
_NW = _NC * _NS
_PW = _S // _NW
_W = 8
_NPC = _PW // _W
_LANES = 16
_UNROLL = 8


def _add_pos(rows, pos):
    @pl.loop(0, _W)
    def _(r):
        @pl.loop(0, _H, step=_LANES * _UNROLL)
        def _(c):
            for u in range(_UNROLL):
                slc = (pl.ds(r, 1), pl.ds(c + _LANES * u, _LANES))
                plsc.addupdate(rows.at[slc], pos[slc])


def _body(x_hbm, tok_hbm, pos_hbm, out_hbm,
          idx_v, pos_v, rows0, rows1, gsem0, gsem1, ssem0, ssem1):
    wid = lax.axis_index("c") * _NS + lax.axis_index("s")
    p0 = wid * _PW
    rows = (rows0, rows1)
    gsem = (gsem0, gsem1)
    ssem = (ssem0, ssem1)

    for b in range(_B):
        pltpu.sync_copy(x_hbm.at[pl.ds(b * _S + p0, _PW)],
                        idx_v.at[pl.ds(b * _PW, _PW)])

    def g_desc(pc, b, buf):
        return pltpu.make_async_copy(
            tok_hbm.at[idx_v.at[pl.ds(b * _PW + pc * _W, _W)]],
            rows[buf], gsem[buf])

    def s_desc(pc, b, buf):
        return pltpu.make_async_copy(
            rows[buf], out_hbm.at[pl.ds(b * _S + p0 + pc * _W, _W)], ssem[buf])

    g_desc(0, 0, 0).start()

    @pl.loop(0, _NPC)
    def _(pc):
        pltpu.sync_copy(pos_hbm.at[pl.ds(p0 + pc * _W, _W)], pos_v)
        for b in range(_B):
            buf = b % 2
            other = 1 - buf
            if b == 0:
                @pl.when(pc > 0)
                def _():
                    s_desc(pc, 3, other).wait()
            else:
                s_desc(pc, b - 1, other).wait()
            if b < _B - 1:
                g_desc(pc, b + 1, other).start()
            else:
                @pl.when(pc < _NPC - 1)
                def _():
                    g_desc(pc + 1, 0, other).start()
            g_desc(pc, b, buf).wait()
            _add_pos(rows[buf], pos_v)
            s_desc(pc, b, buf).start()

    s_desc(_NPC - 1, _B - 1, 1).wait()


_emb_call = pl.kernel(
    _body,
    out_type=jax.ShapeDtypeStruct((_B * _S, _H), jnp.float32),
    mesh=plsc.VectorSubcoreMesh(core_axis_name="c", subcore_axis_name="s"),
    scratch_types=[
        pltpu.VMEM((_B * _PW,), jnp.int32),
        pltpu.VMEM((_W, _H), jnp.float32),
        pltpu.VMEM((_W, _H), jnp.float32),
        pltpu.VMEM((_W, _H), jnp.float32),
        pltpu.SemaphoreType.DMA,
        pltpu.SemaphoreType.DMA,
        pltpu.SemaphoreType.DMA,
        pltpu.SemaphoreType.DMA,
    ],
)


@jax.jit
def _emb(x_flat, tok_emb, pos_emb):
    return _emb_call(x_flat, tok_emb, pos_emb)


def kernel(x, tok_emb, pos_emb):
    x_flat = x.reshape(-1).astype(jnp.int32)
    out = _emb(x_flat, tok_emb, pos_emb)
    return out.reshape(_B, _S, _H)


# speedup vs baseline: 2.2682x; 2.2682x over previous
import jax
import jax.numpy as jnp
from jax import lax
from jax.experimental import pallas as pl
from jax.experimental.pallas import tpu as pltpu
from jax.experimental.pallas import tpu_sc as plsc

_B, _S, _H = 4, 2048, 4096
_NC, _NS = 2, 16
_NW = _NC * _NS
_PW = _S // _NW
_W = 8
_NPC = _PW // _W
_LANES = 16
_UNROLL = 8


def _add_pos(rows, pos):
    @pl.loop(0, _W)
    def _(r):
        @pl.loop(0, _H, step=_LANES * _UNROLL)
        def _(c):
            vals = [pos[r, pl.ds(c + _LANES * u, _LANES)] for u in range(_UNROLL)]
            for u in range(_UNROLL):
                plsc.addupdate(rows.at[r, pl.ds(c + _LANES * u, _LANES)], vals[u])


def _body(x_hbm, tok_hbm, pos_hbm, out_hbm,
          idx_v, pos_v, rows0, rows1, gsem0, gsem1, ssem0, ssem1):
    wid = lax.axis_index("c") * _NS + lax.axis_index("s")
    p0 = wid * _PW
    rows = (rows0, rows1)
    gsem = (gsem0, gsem1)
    ssem = (ssem0, ssem1)

    for b in range(_B):
        pltpu.sync_copy(x_hbm.at[pl.ds(b * _S + p0, _PW)],
                        idx_v.at[pl.ds(b * _PW, _PW)])

    def g_desc(pc, b, buf):
        return pltpu.make_async_copy(
            tok_hbm.at[idx_v.at[pl.ds(b * _PW + pc * _W, _W)]],
            rows[buf], gsem[buf])

    def s_desc(pc, b, buf):
        return pltpu.make_async_copy(
            rows[buf], out_hbm.at[pl.ds(b * _S + p0 + pc * _W, _W)], ssem[buf])

    g_desc(0, 0, 0).start()

    @pl.loop(0, _NPC)
    def _(pc):
        pltpu.sync_copy(pos_hbm.at[pl.ds(p0 + pc * _W, _W)], pos_v)
        for b in range(_B):
            buf = b % 2
            other = 1 - buf
            if b == 0:
                @pl.when(pc > 0)
                def _():
                    s_desc(pc, 3, other).wait()
            else:
                s_desc(pc, b - 1, other).wait()
            if b < _B - 1:
                g_desc(pc, b + 1, other).start()
            else:
                @pl.when(pc < _NPC - 1)
                def _():
                    g_desc(pc + 1, 0, other).start()
            g_desc(pc, b, buf).wait()
            _add_pos(rows[buf], pos_v)
            s_desc(pc, b, buf).start()

    s_desc(_NPC - 1, _B - 1, 1).wait()


_emb_call = pl.kernel(
    _body,
    out_type=jax.ShapeDtypeStruct((_B * _S, _H), jnp.float32),
    mesh=plsc.VectorSubcoreMesh(core_axis_name="c", subcore_axis_name="s"),
    scratch_types=[
        pltpu.VMEM((_B * _PW,), jnp.int32),
        pltpu.VMEM((_W, _H), jnp.float32),
        pltpu.VMEM((_W, _H), jnp.float32),
        pltpu.VMEM((_W, _H), jnp.float32),
        pltpu.SemaphoreType.DMA,
        pltpu.SemaphoreType.DMA,
        pltpu.SemaphoreType.DMA,
        pltpu.SemaphoreType.DMA,
    ],
)


@jax.jit
def _emb(x_flat, tok_emb, pos_emb):
    return _emb_call(x_flat, tok_emb, pos_emb)


def kernel(x, tok_emb, pos_emb):
    x_flat = x.reshape(-1).astype(jnp.int32)
    out = _emb(x_flat, tok_emb, pos_emb)
    return out.reshape(_B, _S, _H)
